# baseline (speedup 1.0000x reference)
import jax
import jax.numpy as jnp
from jax.experimental import pallas as pl
from jax.experimental.pallas import tpu as pltpu

_DB = 256


def _body(q_ref, pos_ref, out_ref, acc_ref):
    b = pl.program_id(1)

    @pl.when(b == 0)
    def _():
        acc_ref[...] = jnp.swapaxes(pos_ref[...], 0, 1)

    out_ref[...] = q_ref[...] + acc_ref[...][None]


def kernel(q, pos_embed):
    bsz, d_model, q_frm = q.shape
    grid = (d_model // _DB, bsz)
    return pl.pallas_call(
        _body,
        grid=grid,
        in_specs=[
            pl.BlockSpec((1, _DB, q_frm), lambda d, b: (b, d, 0)),
            pl.BlockSpec((q_frm, _DB), lambda d, b: (0, d)),
        ],
        out_specs=pl.BlockSpec((1, _DB, q_frm), lambda d, b: (b, d, 0)),
        out_shape=jax.ShapeDtypeStruct((bsz, d_model, q_frm), q.dtype),
        scratch_shapes=[pltpu.VMEM((_DB, q_frm), q.dtype)],
        compiler_params=pltpu.CompilerParams(
            dimension_semantics=("arbitrary", "arbitrary"),
        ),
    )(q, pos_embed)
